# baseline (speedup 1.0000x reference)
import jax
import jax.numpy as jnp
from jax import lax
from jax.experimental import pallas as pl
from jax.experimental.pallas import tpu as pltpu
from jax.experimental.pallas import tpu_sc as plsc

N_ROWS = 1000000
D_FEAT = 64
BATCH = 16384

NC = 2
NS = 16
NW = NC * NS
CHUNK = 128
ROWS_MAJ = BATCH // CHUNK
R_PER_W = ROWS_MAJ // NW


def _gather_kernel(table_hbm, resp_hbm, idx_hbm, out_rows_hbm, out_resp_hbm,
                   idx_v, rows_v, resp_v, sem_rows, sem_resp):
    wid = lax.axis_index("s") * NC + lax.axis_index("c")
    base = wid * R_PER_W

    pltpu.sync_copy(idx_hbm.at[pl.ds(base, R_PER_W)], idx_v)

    handles = []
    for j in range(R_PER_W):
        handles.append(
            pltpu.async_copy(table_hbm.at[idx_v.at[j]], rows_v.at[j], sem_rows))
        handles.append(
            pltpu.async_copy(resp_hbm.at[idx_v.at[j]], resp_v.at[j], sem_resp))
    for h in handles:
        h.wait()

    pltpu.sync_copy(rows_v, out_rows_hbm.at[pl.ds(base, R_PER_W)])
    pltpu.sync_copy(resp_v, out_resp_hbm.at[pl.ds(base, R_PER_W)])


def kernel(input, response, mb_idx):
    idx2d = mb_idx.reshape(ROWS_MAJ, CHUNK)
    mesh = plsc.VectorSubcoreMesh(core_axis_name="c", subcore_axis_name="s")
    out_rows, out_resp = pl.kernel(
        _gather_kernel,
        mesh=mesh,
        compiler_params=pltpu.CompilerParams(use_tc_tiling_on_sc=False),
        out_type=(
            jax.ShapeDtypeStruct((ROWS_MAJ, CHUNK, D_FEAT), jnp.float32),
            jax.ShapeDtypeStruct((ROWS_MAJ, CHUNK), jnp.float32),
        ),
        scratch_types=[
            pltpu.VMEM((R_PER_W, CHUNK), jnp.int32),
            pltpu.VMEM((R_PER_W, CHUNK, D_FEAT), jnp.float32),
            pltpu.VMEM((R_PER_W, CHUNK), jnp.float32),
            pltpu.SemaphoreType.DMA,
            pltpu.SemaphoreType.DMA,
        ],
    )(input, response, idx2d)
    return out_rows.reshape(BATCH, D_FEAT), out_resp.reshape(BATCH)


# speedup vs baseline: 1.7265x; 1.7265x over previous
import jax
import jax.numpy as jnp
from jax import lax
from jax.experimental import pallas as pl
from jax.experimental.pallas import tpu as pltpu
from jax.experimental.pallas import tpu_sc as plsc

N_ROWS = 1000000
D_FEAT = 64
BATCH = 16384

NC = 2
NS = 16
NW = NC * NS
B_PER_W = BATCH // NW
NG = B_PER_W // 16


def _gather_kernel(table_hbm, resp_hbm, idx_hbm, out_rows_hbm, out_resp_hbm,
                   idx_v, rows_v, resp16_v, resp_v, sem_rows, sem_resp):
    wid = lax.axis_index("s") * NC + lax.axis_index("c")
    base = wid * B_PER_W

    pltpu.sync_copy(idx_hbm.at[pl.ds(base, B_PER_W)], idx_v)

    def fire(g, _):
        idx_vec = idx_v[pl.ds(pl.multiple_of(g * 16, 8), 16)]
        for j in range(16):
            k = g * 16 + j
            r = idx_vec[j]
            r16 = pl.multiple_of(
                lax.shift_left(lax.shift_right_logical(r, 4), 4), 8)
            pltpu.async_copy(table_hbm.at[pl.ds(r, 1)],
                             rows_v.at[pl.ds(k, 1)], sem_rows)
            pltpu.async_copy(resp_hbm.at[pl.ds(r16, 16)],
                             resp16_v.at[pl.ds(pl.multiple_of(k * 16, 8), 16)],
                             sem_resp)
        return _

    lax.fori_loop(0, NG, fire, None)

    pltpu.make_async_copy(table_hbm.at[pl.ds(0, B_PER_W)], rows_v,
                          sem_rows).wait()
    pltpu.make_async_copy(resp_hbm.at[pl.ds(0, 16 * B_PER_W)], resp16_v,
                          sem_resp).wait()

    lanes = lax.iota(jnp.int32, 16)

    def pick(g, _):
        idx_vec = idx_v[pl.ds(pl.multiple_of(g * 16, 8), 16)]
        acc = jnp.zeros((16,), jnp.float32)
        for j in range(16):
            k = g * 16 + j
            grp = resp16_v[pl.ds(pl.multiple_of(k * 16, 8), 16)]
            sub = lax.bitwise_and(idx_vec[j], 15)
            val = jnp.take_along_axis(
                grp, lax.broadcast(sub, (16,)), axis=0,
                mode=lax.GatherScatterMode.PROMISE_IN_BOUNDS)
            acc = jnp.where(lanes == j, val, acc)
        resp_v[pl.ds(pl.multiple_of(g * 16, 8), 16)] = acc
        return _

    lax.fori_loop(0, NG, pick, None)

    pltpu.sync_copy(rows_v, out_rows_hbm.at[pl.ds(base, B_PER_W)])
    pltpu.sync_copy(resp_v, out_resp_hbm.at[pl.ds(base, B_PER_W)])


def kernel(input, response, mb_idx):
    mesh = plsc.VectorSubcoreMesh(core_axis_name="c", subcore_axis_name="s")
    out_rows, out_resp = pl.kernel(
        _gather_kernel,
        mesh=mesh,
        out_type=(
            jax.ShapeDtypeStruct((BATCH, D_FEAT), jnp.float32),
            jax.ShapeDtypeStruct((BATCH,), jnp.float32),
        ),
        scratch_types=[
            pltpu.VMEM((B_PER_W,), jnp.int32),
            pltpu.VMEM((B_PER_W, D_FEAT), jnp.float32),
            pltpu.VMEM((16 * B_PER_W,), jnp.float32),
            pltpu.VMEM((B_PER_W,), jnp.float32),
            pltpu.SemaphoreType.DMA,
            pltpu.SemaphoreType.DMA,
        ],
    )(input, response, mb_idx)
    return out_rows, out_resp
